# Initial kernel scaffold; baseline (speedup 1.0000x reference)
import jax
import jax.numpy as jnp
from jax.experimental import pallas as pl


def kernel(anchors, objectness, box_regression):
    raise NotImplementedError("write your pallas kernel here")


# speedup vs baseline: 2.6129x; 2.6129x over previous
import math

import jax
import jax.numpy as jnp
from jax.experimental import pallas as pl
from jax.experimental.pallas import tpu as pltpu

_N, _A, _H, _W = 2, 3, 160, 160
_NUM = _A * _H * _W
_ROWS, _LANES = 600, 128
_PRE = 6000
_POST = 1000
_THRESH = 0.7
_IM_W, _IM_H = 800.0, 800.0
_MIN_SIZE = 0.0
_BBOX_CLIP = float(math.log(1000.0 / 16.0))


def _rpn_kernel(logit_ref, anc_ref, reg_ref,
                boxes_ref, scores_ref, mask_ref,
                x1_ref, y1_ref, x2_ref, y2_ref, area_ref, s_ref):
    logit = logit_ref[0]
    score = jax.nn.sigmoid(logit)

    row_iota = jax.lax.broadcasted_iota(jnp.int32, (_ROWS, _LANES), 0)
    col_iota = jax.lax.broadcasted_iota(jnp.int32, (_ROWS, _LANES), 1)
    iota = row_iota * _LANES + col_iota

    def _bis_body(_, carry):
        lo, hi = carry
        mid = 0.5 * (lo + hi)
        cnt = jnp.sum((score >= mid).astype(jnp.int32))
        take = cnt >= _PRE
        return jnp.where(take, mid, lo), jnp.where(take, hi, mid)

    lo, hi = jax.lax.fori_loop(
        0, 60, _bis_body, (jnp.float32(0.0), jnp.float32(1.0)))

    n_hi = jnp.sum((score >= hi).astype(jnp.int32))
    k = _PRE - n_hi
    ties = (score >= lo) & (score < hi)

    def _tie_body(_, carry):
        lo_t, hi_t = carry
        mid_t = (lo_t + hi_t) // 2
        cnt = jnp.sum((ties & (iota <= mid_t)).astype(jnp.int32))
        take = cnt >= k
        return jnp.where(take, lo_t, mid_t + 1), jnp.where(take, mid_t, hi_t)

    _, tie_T = jax.lax.fori_loop(
        0, 18, _tie_body, (jnp.int32(0), jnp.int32(_NUM - 1)))

    participate = (score >= hi) | (ties & (iota <= tie_T))
    s0 = jnp.where(participate, score, -1.0)

    ax1 = anc_ref[0, 0]
    ay1 = anc_ref[0, 1]
    ax2 = anc_ref[0, 2]
    ay2 = anc_ref[0, 3]
    dx = reg_ref[0, 0]
    dy = reg_ref[0, 1]
    dw = jnp.minimum(reg_ref[0, 2], _BBOX_CLIP)
    dh = jnp.minimum(reg_ref[0, 3], _BBOX_CLIP)

    widths = ax2 - ax1 + 1.0
    heights = ay2 - ay1 + 1.0
    ctr_x = ax1 + 0.5 * widths
    ctr_y = ay1 + 0.5 * heights
    pred_ctr_x = dx * widths + ctr_x
    pred_ctr_y = dy * heights + ctr_y
    pred_w = jnp.exp(dw) * widths
    pred_h = jnp.exp(dh) * heights

    x1 = jnp.clip(pred_ctr_x - 0.5 * pred_w, 0.0, _IM_W - 1.0)
    y1 = jnp.clip(pred_ctr_y - 0.5 * pred_h, 0.0, _IM_H - 1.0)
    x2 = jnp.clip(pred_ctr_x + 0.5 * pred_w - 1.0, 0.0, _IM_W - 1.0)
    y2 = jnp.clip(pred_ctr_y + 0.5 * pred_h - 1.0, 0.0, _IM_H - 1.0)

    ws = x2 - x1 + 1.0
    hs = y2 - y1 + 1.0
    keep = (ws >= _MIN_SIZE) & (hs >= _MIN_SIZE)
    s0 = jnp.where(keep, s0, -1.0)

    x1_ref[...] = x1
    y1_ref[...] = y1
    x2_ref[...] = x2
    y2_ref[...] = y2
    area_ref[...] = ws * hs
    s_ref[...] = s0

    lane_iota = jax.lax.broadcasted_iota(jnp.int32, (1, _LANES), 1)

    def _nms_body(i, _):
        s = s_ref[...]
        m = jnp.max(s)
        sel = jnp.min(jnp.where(s == m, iota, _NUM))
        valid = m > 0.0
        r = sel // _LANES
        c = sel - r * _LANES

        def _extract(ref):
            row = ref[pl.ds(r, 1), :]
            return jnp.sum(jnp.where(lane_iota == c, row, 0.0))

        bx1 = _extract(x1_ref)
        by1 = _extract(y1_ref)
        bx2 = _extract(x2_ref)
        by2 = _extract(y2_ref)
        barea = _extract(area_ref)

        xx1 = jnp.maximum(bx1, x1_ref[...])
        yy1 = jnp.maximum(by1, y1_ref[...])
        xx2 = jnp.minimum(bx2, x2_ref[...])
        yy2 = jnp.minimum(by2, y2_ref[...])
        w = jnp.maximum(xx2 - xx1 + 1.0, 0.0)
        h = jnp.maximum(yy2 - yy1 + 1.0, 0.0)
        inter = w * h
        iou = inter / (barea + area_ref[...] - inter)

        s_new = jnp.where(iou > _THRESH, -1.0, s)
        s_new = jnp.where(iota == sel, -1.0, s_new)
        s_ref[...] = jnp.where(valid, s_new, s)

        zero = jnp.float32(0.0)
        ci4 = jax.lax.broadcasted_iota(jnp.int32, (1, 4), 1)
        box_row = jnp.where(ci4 == 0, jnp.where(valid, bx1, zero),
                  jnp.where(ci4 == 1, jnp.where(valid, by1, zero),
                  jnp.where(ci4 == 2, jnp.where(valid, bx2, zero),
                            jnp.where(valid, by2, zero))))
        boxes_ref[0, pl.ds(i, 1), :] = box_row
        scores_ref[0, pl.ds(i, 1), :] = jnp.full(
            (1, 1), jnp.where(valid, m, zero), dtype=jnp.float32)
        mask_ref[0, pl.ds(i, 1), :] = jnp.full(
            (1, 1), jnp.where(valid, jnp.float32(1.0), zero), dtype=jnp.float32)
        return 0

    jax.lax.fori_loop(0, _POST, _nms_body, 0)


def kernel(anchors, objectness, box_regression):
    obj = jnp.transpose(objectness, (0, 2, 3, 1)).reshape(_N, _ROWS, _LANES)
    reg = box_regression.reshape(_N, _A, 4, _H, _W)
    reg = jnp.transpose(reg, (0, 3, 4, 1, 2)).reshape(_N, _NUM, 4)
    reg = jnp.transpose(reg, (0, 2, 1)).reshape(_N, 4, _ROWS, _LANES)
    anc = jnp.transpose(anchors.reshape(_N, _NUM, 4), (0, 2, 1))
    anc = anc.reshape(_N, 4, _ROWS, _LANES)

    boxes, scores, mask = pl.pallas_call(
        _rpn_kernel,
        grid=(_N,),
        in_specs=[
            pl.BlockSpec((1, _ROWS, _LANES), lambda n: (n, 0, 0)),
            pl.BlockSpec((1, 4, _ROWS, _LANES), lambda n: (n, 0, 0, 0)),
            pl.BlockSpec((1, 4, _ROWS, _LANES), lambda n: (n, 0, 0, 0)),
        ],
        out_specs=[
            pl.BlockSpec((1, _POST, 4), lambda n: (n, 0, 0)),
            pl.BlockSpec((1, _POST, 1), lambda n: (n, 0, 0)),
            pl.BlockSpec((1, _POST, 1), lambda n: (n, 0, 0)),
        ],
        out_shape=[
            jax.ShapeDtypeStruct((_N, _POST, 4), jnp.float32),
            jax.ShapeDtypeStruct((_N, _POST, 1), jnp.float32),
            jax.ShapeDtypeStruct((_N, _POST, 1), jnp.float32),
        ],
        scratch_shapes=[pltpu.VMEM((_ROWS, _LANES), jnp.float32)
                        for _ in range(6)],
        compiler_params=pltpu.CompilerParams(
            dimension_semantics=("arbitrary",)),
    )(obj, anc, reg)

    return boxes, scores.reshape(_N, _POST), mask.reshape(_N, _POST)
